# baseline (speedup 1.0000x reference)
import functools

import jax
import jax.numpy as jnp
from jax import lax
from jax.experimental import pallas as pl
from jax.experimental.pallas import tpu as pltpu
from jax.experimental.pallas import tpu_sc as plsc

_NC = 2
_NS = 16
_NW = _NC * _NS
_R = 8
_L = 16


def _sc_body(x_hbm, pos_hbm, o_hbm, buf0, buf1, pos_v, psem, sem0, sem1, osem0, osem1):
    wid = lax.axis_index("s") * _NC + lax.axis_index("c")
    rows_w = x_hbm.shape[0] // _NW
    base = wid * rows_w
    n_chunks = rows_w // _R
    row_len = x_hbm.shape[1]
    nv = row_len // _L

    pltpu.async_copy(pos_hbm.at[pl.ds(base, rows_w)], pos_v, psem).wait()

    def in_copy(g, buf, sem):
        return pltpu.make_async_copy(
            x_hbm.at[pl.ds(base + g * _R, _R)], buf, sem
        )

    def out_copy(g, buf, sem):
        return pltpu.make_async_copy(
            buf, o_hbm.at[pl.ds(base + g * _R, _R)], sem
        )

    in_copy(0, buf0, sem0).start()

    bufs = (buf0, buf1)
    isems = (sem0, sem1)
    osems = (osem0, osem1)

    def do_chunk(g, slot):
        buf, isem, osem = bufs[slot], isems[slot], osems[slot]
        in_copy(g, buf, isem).wait()

        @pl.when(g >= 2)
        def _():
            out_copy(g - 2, buf, osem).wait()

        for r in range(_R):
            pv = pos_v[g * _R + r, :]

            @plsc.parallel_loop(0, nv, 1, unroll=8)
            def _add_row(j, r=r, pv=pv, buf=buf):
                sl = pl.ds(j * _L, _L)
                buf[r, sl] = buf[r, sl] + pv

        out_copy(g, buf, osem).start()

        @pl.when(g + 1 < n_chunks)
        def _():
            in_copy(g + 1, bufs[1 - slot], isems[1 - slot]).start()

    def pair(g2, carry):
        do_chunk(g2, 0)
        do_chunk(g2 + 1, 1)
        return carry

    lax.fori_loop(0, n_chunks // 2, lambda k, c: pair(k * 2, c), 0)

    out_copy(n_chunks - 2, buf0, osem0).wait()
    out_copy(n_chunks - 1, buf1, osem1).wait()


def kernel(x, pos_emb):
    B, S, D = x.shape
    xt = jnp.transpose(x, (1, 2, 0)).reshape(S * D, B)
    pos_flat = jnp.broadcast_to(pos_emb.reshape(S * D, 1), (S * D, _L))
    mesh = plsc.VectorSubcoreMesh(core_axis_name="c", subcore_axis_name="s")
    f = functools.partial(
        pl.kernel,
        mesh=mesh,
        out_type=jax.ShapeDtypeStruct((S * D, B), x.dtype),
        scratch_types=[
            pltpu.VMEM((_R, B), x.dtype),
            pltpu.VMEM((_R, B), x.dtype),
            pltpu.VMEM((S * D // _NW, _L), x.dtype),
            pltpu.SemaphoreType.DMA,
            pltpu.SemaphoreType.DMA,
            pltpu.SemaphoreType.DMA,
            pltpu.SemaphoreType.DMA,
            pltpu.SemaphoreType.DMA,
        ],
    )(_sc_body)
    out_t = f(xt, pos_flat)
    return jnp.transpose(out_t.reshape(S, D, B), (2, 0, 1))


# speedup vs baseline: 1.2601x; 1.2601x over previous
import functools

import jax
import jax.numpy as jnp
from jax import lax
from jax.experimental import pallas as pl
from jax.experimental.pallas import tpu as pltpu
from jax.experimental.pallas import tpu_sc as plsc

_NC = 2
_NS = 16
_NW = _NC * _NS
_R = 8
_L = 16


def _sc_body(x_hbm, pos_hbm, o_hbm, buf0, buf1, pos_v, psem, sem0, sem1, osem0, osem1):
    wid = lax.axis_index("s") * _NC + lax.axis_index("c")
    rows_w = x_hbm.shape[0] // _NW
    base = wid * rows_w
    n_chunks = rows_w // _R
    row_len = x_hbm.shape[1]
    nv = row_len // _L

    pltpu.async_copy(pos_hbm.at[pl.ds(base, rows_w)], pos_v, psem).wait()

    def in_copy(g, buf, sem):
        return pltpu.make_async_copy(
            x_hbm.at[pl.ds(base + g * _R, _R)], buf, sem
        )

    def out_copy(g, buf, sem):
        return pltpu.make_async_copy(
            buf, o_hbm.at[pl.ds(base + g * _R, _R)], sem
        )

    in_copy(0, buf0, sem0).start()

    bufs = (buf0, buf1)
    isems = (sem0, sem1)
    osems = (osem0, osem1)

    def do_chunk(g, slot):
        buf, isem, osem = bufs[slot], isems[slot], osems[slot]
        obuf, oisem, oosem = bufs[1 - slot], isems[1 - slot], osems[1 - slot]
        in_copy(g, buf, isem).wait()

        @pl.when(g >= 1)
        def _():
            out_copy(g - 1, obuf, oosem).wait()

        @pl.when(g + 1 < n_chunks)
        def _():
            in_copy(g + 1, obuf, oisem).start()

        for r in range(_R):
            pv = pos_v[g * _R + r, :]

            @plsc.parallel_loop(0, nv, 1, unroll=8)
            def _add_row(j, r=r, pv=pv, buf=buf):
                sl = pl.ds(j * _L, _L)
                buf[r, sl] = buf[r, sl] + pv

        out_copy(g, buf, osem).start()

    def pair(g2, carry):
        do_chunk(g2, 0)
        do_chunk(g2 + 1, 1)
        return carry

    lax.fori_loop(0, n_chunks // 2, lambda k, c: pair(k * 2, c), 0)

    last = n_chunks - 1
    out_copy(last, bufs[last % 2], osems[last % 2]).wait()


def kernel(x, pos_emb):
    B, S, D = x.shape
    xt = jnp.transpose(x, (1, 2, 0)).reshape(S * D, B)
    pos_flat = jnp.broadcast_to(pos_emb.reshape(S * D, 1), (S * D, _L))
    mesh = plsc.VectorSubcoreMesh(core_axis_name="c", subcore_axis_name="s")
    f = functools.partial(
        pl.kernel,
        mesh=mesh,
        out_type=jax.ShapeDtypeStruct((S * D, B), x.dtype),
        scratch_types=[
            pltpu.VMEM((_R, B), x.dtype),
            pltpu.VMEM((_R, B), x.dtype),
            pltpu.VMEM((S * D // _NW, _L), x.dtype),
            pltpu.SemaphoreType.DMA,
            pltpu.SemaphoreType.DMA,
            pltpu.SemaphoreType.DMA,
            pltpu.SemaphoreType.DMA,
            pltpu.SemaphoreType.DMA,
        ],
    )(_sc_body)
    out_t = f(xt, pos_flat)
    return jnp.transpose(out_t.reshape(S, D, B), (2, 0, 1))


# speedup vs baseline: 1.2686x; 1.0068x over previous
import functools

import jax
import jax.numpy as jnp
from jax import lax
from jax.experimental import pallas as pl
from jax.experimental.pallas import tpu as pltpu
from jax.experimental.pallas import tpu_sc as plsc

_NC = 2
_NS = 16
_NW = _NC * _NS
_R = 8
_CW = 2048
_L = 16


def _sc_body(x_hbm, pos_hbm, o_hbm, ibuf0, ibuf1, obuf0, obuf1, pos_v,
             psem, isem0, isem1, osem0, osem1):
    wid = lax.axis_index("s") * _NC + lax.axis_index("c")
    rows_w = x_hbm.shape[0] // _NW
    base = wid * rows_w
    row_len = x_hbm.shape[1]
    splits = row_len // _CW
    n_chunks = (rows_w // _R) * splits
    nv = _CW // _L

    pltpu.async_copy(pos_hbm.at[pl.ds(base, rows_w)], pos_v, psem).wait()

    ibufs = (ibuf0, ibuf1)
    obufs = (obuf0, obuf1)
    isems = (isem0, isem1)
    osems = (osem0, osem1)

    def rc(g):
        return base + lax.div(g, splits) * _R, lax.rem(g, splits) * _CW

    def in_copy(g, slot):
        r0, c0 = rc(g)
        return pltpu.make_async_copy(
            x_hbm.at[pl.ds(r0, _R), pl.ds(c0, _CW)], ibufs[slot], isems[slot]
        )

    def out_copy(g, slot):
        r0, c0 = rc(g)
        return pltpu.make_async_copy(
            obufs[slot], o_hbm.at[pl.ds(r0, _R), pl.ds(c0, _CW)], osems[slot]
        )

    in_copy(0, 0).start()
    in_copy(1, 1).start()

    def do_chunk(g, slot):
        ibuf, obuf = ibufs[slot], obufs[slot]
        in_copy(g, slot).wait()

        @pl.when(g >= 2)
        def _():
            out_copy(g - 2, slot).wait()

        prow = lax.div(g, splits) * _R
        for r in range(_R):
            pv = pos_v[prow + r, :]

            @plsc.parallel_loop(0, nv, 1, unroll=8)
            def _add_row(j, r=r, pv=pv, ibuf=ibuf, obuf=obuf):
                sl = pl.ds(j * _L, _L)
                obuf[r, sl] = ibuf[r, sl] + pv

        out_copy(g, slot).start()

        @pl.when(g + 2 < n_chunks)
        def _():
            in_copy(g + 2, slot).start()

    def pair(g2, carry):
        do_chunk(g2, 0)
        do_chunk(g2 + 1, 1)
        return carry

    lax.fori_loop(0, n_chunks // 2, lambda k, c: pair(k * 2, c), 0)

    out_copy(n_chunks - 2, 0).wait()
    out_copy(n_chunks - 1, 1).wait()


def kernel(x, pos_emb):
    B, S, D = x.shape
    xt = jnp.transpose(x, (1, 2, 0)).reshape(S * D, B)
    pos_flat = jnp.broadcast_to(pos_emb.reshape(S * D, 1), (S * D, _L))
    mesh = plsc.VectorSubcoreMesh(core_axis_name="c", subcore_axis_name="s")
    f = functools.partial(
        pl.kernel,
        mesh=mesh,
        out_type=jax.ShapeDtypeStruct((S * D, B), x.dtype),
        scratch_types=[
            pltpu.VMEM((_R, _CW), x.dtype),
            pltpu.VMEM((_R, _CW), x.dtype),
            pltpu.VMEM((_R, _CW), x.dtype),
            pltpu.VMEM((_R, _CW), x.dtype),
            pltpu.VMEM((S * D // _NW, _L), x.dtype),
            pltpu.SemaphoreType.DMA,
            pltpu.SemaphoreType.DMA,
            pltpu.SemaphoreType.DMA,
            pltpu.SemaphoreType.DMA,
            pltpu.SemaphoreType.DMA,
        ],
    )(_sc_body)
    out_t = f(xt, pos_flat)
    return jnp.transpose(out_t.reshape(S, D, B), (2, 0, 1))


# speedup vs baseline: 1.2737x; 1.0041x over previous
import functools

import jax
import jax.numpy as jnp
from jax import lax
from jax.experimental import pallas as pl
from jax.experimental.pallas import tpu as pltpu
from jax.experimental.pallas import tpu_sc as plsc

_NC = 2
_NS = 16
_NW = _NC * _NS
_R = 8
_CW = 2048
_L = 16


def _sc_body(x_hbm, pos_hbm, o_hbm, ibuf0, ibuf1, obuf0, obuf1, pos_v,
             psem, isem0, isem1, osem0, osem1):
    wid = lax.axis_index("s") * _NC + lax.axis_index("c")
    rows_w = x_hbm.shape[0] // _NW
    base = wid * rows_w
    row_len = x_hbm.shape[1]
    splits = row_len // _CW
    n_chunks = (rows_w // _R) * splits
    nv = _CW // _L

    pltpu.async_copy(pos_hbm.at[pl.ds(base, rows_w)], pos_v, psem).wait()

    ibufs = (ibuf0, ibuf1)
    obufs = (obuf0, obuf1)
    isems = (isem0, isem1)
    osems = (osem0, osem1)

    def rc(g):
        return base + lax.div(g, splits) * _R, lax.rem(g, splits) * _CW

    def in_copy(g, slot):
        r0, c0 = rc(g)
        return pltpu.make_async_copy(
            x_hbm.at[pl.ds(r0, _R), pl.ds(c0, _CW)], ibufs[slot], isems[slot]
        )

    def out_copy(g, slot):
        r0, c0 = rc(g)
        return pltpu.make_async_copy(
            ibufs[slot], o_hbm.at[pl.ds(r0, _R), pl.ds(c0, _CW)], osems[slot]
        )

    in_copy(0, 0).start()
    in_copy(1, 1).start()

    def do_chunk(g, slot):
        ibuf, obuf = ibufs[slot], obufs[slot]
        in_copy(g, slot).wait()

        @pl.when(g >= 2)
        def _():
            out_copy(g - 2, slot).wait()

        out_copy(g, slot).start()

        @pl.when(g + 2 < n_chunks)
        def _():
            in_copy(g + 2, slot).start()

    def pair(g2, carry):
        do_chunk(g2, 0)
        do_chunk(g2 + 1, 1)
        return carry

    lax.fori_loop(0, n_chunks // 2, lambda k, c: pair(k * 2, c), 0)

    out_copy(n_chunks - 2, 0).wait()
    out_copy(n_chunks - 1, 1).wait()


def kernel(x, pos_emb):
    B, S, D = x.shape
    xt = jnp.transpose(x, (1, 2, 0)).reshape(S * D, B)
    pos_flat = jnp.broadcast_to(pos_emb.reshape(S * D, 1), (S * D, _L))
    mesh = plsc.VectorSubcoreMesh(core_axis_name="c", subcore_axis_name="s")
    f = functools.partial(
        pl.kernel,
        mesh=mesh,
        out_type=jax.ShapeDtypeStruct((S * D, B), x.dtype),
        scratch_types=[
            pltpu.VMEM((_R, _CW), x.dtype),
            pltpu.VMEM((_R, _CW), x.dtype),
            pltpu.VMEM((_R, _CW), x.dtype),
            pltpu.VMEM((_R, _CW), x.dtype),
            pltpu.VMEM((S * D // _NW, _L), x.dtype),
            pltpu.SemaphoreType.DMA,
            pltpu.SemaphoreType.DMA,
            pltpu.SemaphoreType.DMA,
            pltpu.SemaphoreType.DMA,
            pltpu.SemaphoreType.DMA,
        ],
    )(_sc_body)
    out_t = f(xt, pos_flat)
    return jnp.transpose(out_t.reshape(S, D, B), (2, 0, 1))


# speedup vs baseline: 1.7108x; 1.3432x over previous
import jax
import jax.numpy as jnp
from jax.experimental import pallas as pl


def _make_body(SB, D, B):
    def _add_body(x_ref, p_ref, o_ref):
        p = jax.lax.broadcast_in_dim(p_ref[...], (SB, D, B), (0, 1))
        o_ref[...] = x_ref[...] + p
    return _add_body


def kernel(x, pos_emb):
    B, S, D = x.shape
    xt = jnp.transpose(x, (1, 2, 0))
    SB = 8
    out_t = pl.pallas_call(
        _make_body(SB, D, B),
        grid=(S // SB,),
        in_specs=[
            pl.BlockSpec((SB, D, B), lambda i: (i, 0, 0)),
            pl.BlockSpec((SB, D), lambda i: (i, 0)),
        ],
        out_specs=pl.BlockSpec((SB, D, B), lambda i: (i, 0, 0)),
        out_shape=jax.ShapeDtypeStruct((S, D, B), x.dtype),
    )(xt, pos_emb)
    return jnp.transpose(out_t, (2, 0, 1))
